# Initial kernel scaffold; baseline (speedup 1.0000x reference)
import jax
import jax.numpy as jnp
from jax.experimental import pallas as pl


def kernel(q_tensor, k_tensor, v_tensor):
    raise NotImplementedError("write your pallas kernel here")


# speedup vs baseline: 5.6912x; 5.6912x over previous
import functools
import math

import jax
import jax.numpy as jnp
from jax.experimental import pallas as pl

SINK = 4
QMAX = 127.0
K_OUT_KEYS = 8
K_OUT_VALS = 2
EPS = 1e-8
NEG = -1e30


def _recon_body(k_ref, v_ref, kr_ref, vr_ref):
    k = k_ref[0]
    v = v_ref[0]
    s, d = k.shape
    s_iota = jax.lax.broadcasted_iota(jnp.int32, (s, d), 0)
    d_iota = jax.lax.broadcasted_iota(jnp.int32, (s, d), 1)
    sink = s_iota < SINK

    work = jnp.where(sink, 0.0, jnp.abs(k))
    omask = jnp.zeros((s, d), dtype=jnp.bool_)
    for _ in range(K_OUT_KEYS):
        colmax = jnp.max(work, axis=0, keepdims=True)
        first = jnp.min(jnp.where(work == colmax, s_iota, s), axis=0, keepdims=True)
        hit = s_iota == first
        omask = jnp.logical_or(omask, hit)
        work = jnp.where(hit, 0.0, work)
    k_keep = jnp.logical_or(sink, omask)
    k_scale = jnp.maximum(jnp.max(work, axis=0, keepdims=True), EPS) / QMAX
    k_q = jnp.clip(jnp.round(jnp.where(k_keep, 0.0, k) / k_scale), -QMAX, QMAX)
    kr_ref[0] = jnp.where(k_keep, k, k_q * k_scale)

    workv = jnp.where(sink, 0.0, jnp.abs(v))
    vmask = jnp.zeros((s, d), dtype=jnp.bool_)
    for _ in range(K_OUT_VALS):
        rowmax = jnp.max(workv, axis=1, keepdims=True)
        first = jnp.min(jnp.where(workv == rowmax, d_iota, d), axis=1, keepdims=True)
        hit = d_iota == first
        vmask = jnp.logical_or(vmask, hit)
        workv = jnp.where(hit, 0.0, workv)
    v_keep = jnp.logical_or(sink, vmask)
    v_scale = jnp.maximum(jnp.max(workv, axis=1, keepdims=True), EPS) / QMAX
    v_q = jnp.clip(jnp.round(jnp.where(v_keep, 0.0, v) / v_scale), -QMAX, QMAX)
    vr_ref[0] = jnp.where(v_keep, v, v_q * v_scale)


def _attn_body(q_ref, k_ref, v_ref, o_ref, *, bq):
    qb = q_ref[0]
    k = k_ref[0]
    v = v_ref[0]
    s, d = k.shape
    i = pl.program_id(1)
    row = i * bq + jax.lax.broadcasted_iota(jnp.int32, (bq, s), 0)
    col = jax.lax.broadcasted_iota(jnp.int32, (bq, s), 1)
    scores = jax.lax.dot_general(
        qb, k, (((1,), (1,)), ((), ())), preferred_element_type=jnp.float32
    ) * (1.0 / math.sqrt(d))
    scores = jnp.where(col <= row, scores, NEG)
    m = jnp.max(scores, axis=1, keepdims=True)
    p = jnp.exp(scores - m)
    l = jnp.sum(p, axis=1, keepdims=True)
    acc = jax.lax.dot_general(
        p, v, (((1,), (0,)), ((), ())), preferred_element_type=jnp.float32
    )
    o_ref[0] = acc / l


def kernel(q_tensor, k_tensor, v_tensor):
    b, h, s, d = q_tensor.shape
    q = q_tensor.reshape(b * h, s, d)
    k = k_tensor.reshape(b * h, s, d)
    v = v_tensor.reshape(b * h, s, d)
    kr, vr = pl.pallas_call(
        _recon_body,
        grid=(b * h,),
        in_specs=[
            pl.BlockSpec((1, s, d), lambda i: (i, 0, 0)),
            pl.BlockSpec((1, s, d), lambda i: (i, 0, 0)),
        ],
        out_specs=[
            pl.BlockSpec((1, s, d), lambda i: (i, 0, 0)),
            pl.BlockSpec((1, s, d), lambda i: (i, 0, 0)),
        ],
        out_shape=[
            jax.ShapeDtypeStruct((b * h, s, d), jnp.float32),
            jax.ShapeDtypeStruct((b * h, s, d), jnp.float32),
        ],
    )(k, v)
    bq = 512
    out = pl.pallas_call(
        functools.partial(_attn_body, bq=bq),
        grid=(b * h, s // bq),
        in_specs=[
            pl.BlockSpec((1, bq, d), lambda i, j: (i, j, 0)),
            pl.BlockSpec((1, s, d), lambda i, j: (i, 0, 0)),
            pl.BlockSpec((1, s, d), lambda i, j: (i, 0, 0)),
        ],
        out_specs=pl.BlockSpec((1, bq, d), lambda i, j: (i, j, 0)),
        out_shape=jax.ShapeDtypeStruct((b * h, s, d), jnp.float32),
    )(q, kr, vr)
    return out.reshape(b, h, s, d)


# speedup vs baseline: 10.6709x; 1.8750x over previous
import math

import jax
import jax.numpy as jnp
from jax.experimental import pallas as pl
from jax.experimental.pallas import tpu as pltpu

SINK = 4
QMAX = 127.0
K_OUT_KEYS = 8
K_OUT_VALS = 2
EPS = 1e-8
NEG = -1e30
BQ = 512


def _body(q_ref, k_ref, v_ref, o_ref):
    k = k_ref[0, 0]
    v = v_ref[0, 0]
    s, d = k.shape
    s_iota = jax.lax.broadcasted_iota(jnp.int32, (s, d), 0)
    d_iota = jax.lax.broadcasted_iota(jnp.int32, (s, d), 1)
    sink = s_iota < SINK

    work = jnp.where(sink, 0.0, jnp.abs(k))
    for _ in range(K_OUT_KEYS):
        colmax = jnp.max(work, axis=0, keepdims=True)
        first = jnp.min(jnp.where(work == colmax, s_iota, s), axis=0, keepdims=True)
        work = jnp.where(s_iota == first, 0.0, work)
    k_scale = jnp.maximum(jnp.max(work, axis=0, keepdims=True), EPS) / QMAX
    k_q = jnp.clip(jnp.round(work / k_scale * jnp.sign(k)), -QMAX, QMAX)
    k_rec = jnp.where(work == 0.0, k, k_q * k_scale)

    workv = jnp.where(sink, 0.0, jnp.abs(v))
    for _ in range(K_OUT_VALS):
        rowmax = jnp.max(workv, axis=1, keepdims=True)
        first = jnp.min(jnp.where(workv == rowmax, d_iota, d), axis=1, keepdims=True)
        workv = jnp.where(d_iota == first, 0.0, workv)
    v_scale = jnp.maximum(jnp.max(workv, axis=1, keepdims=True), EPS) / QMAX
    v_q = jnp.clip(jnp.round(workv / v_scale * jnp.sign(v)), -QMAX, QMAX)
    v_rec = jnp.where(workv == 0.0, v, v_q * v_scale)

    inv_sqrt_d = 1.0 / math.sqrt(d)
    for i in range(s // BQ):
        span = (i + 1) * BQ
        qb = q_ref[0, 0, i * BQ:span, :]
        kb = k_rec[:span, :]
        vb = v_rec[:span, :]
        row = i * BQ + jax.lax.broadcasted_iota(jnp.int32, (BQ, span), 0)
        col = jax.lax.broadcasted_iota(jnp.int32, (BQ, span), 1)
        scores = jax.lax.dot_general(
            qb, kb, (((1,), (1,)), ((), ())), preferred_element_type=jnp.float32
        ) * inv_sqrt_d
        scores = jnp.where(col <= row, scores, NEG)
        m = jnp.max(scores, axis=1, keepdims=True)
        p = jnp.exp(scores - m)
        l = jnp.sum(p, axis=1, keepdims=True)
        acc = jax.lax.dot_general(
            p, vb, (((1,), (0,)), ((), ())), preferred_element_type=jnp.float32
        )
        o_ref[0, 0, i * BQ:span, :] = acc / l


def kernel(q_tensor, k_tensor, v_tensor):
    b, h, s, d = q_tensor.shape
    spec = pl.BlockSpec((1, 1, s, d), lambda i: (0, i, 0, 0))
    out = pl.pallas_call(
        _body,
        grid=(b * h,),
        in_specs=[spec, spec, spec],
        out_specs=spec,
        out_shape=jax.ShapeDtypeStruct((b, h, s, d), jnp.float32),
        compiler_params=pltpu.CompilerParams(
            dimension_semantics=("parallel",),
        ),
    )(q_tensor, k_tensor, v_tensor)
    return out


# speedup vs baseline: 12.8850x; 1.2075x over previous
import math

import jax
import jax.numpy as jnp
from jax.experimental import pallas as pl
from jax.experimental.pallas import tpu as pltpu

SINK = 4
QMAX = 127.0
K_OUT_KEYS = 8
K_OUT_VALS = 2
EPS = 1e-8
NEG = -1e30
BQ = 512


def _body(q_ref, k_ref, v_ref, o_ref):
    k = k_ref[0, 0]
    v = v_ref[0, 0]
    s, d = k.shape
    s_iota = jax.lax.broadcasted_iota(jnp.int32, (s, d), 0)
    d_iota = jax.lax.broadcasted_iota(jnp.int32, (s, d), 1)
    sink = s_iota < SINK

    work = jnp.where(sink, 0.0, jnp.abs(k))
    for _ in range(K_OUT_KEYS):
        colmax = jnp.max(work, axis=0, keepdims=True)
        work = jnp.where(work == colmax, 0.0, work)
    k_scale = jnp.maximum(jnp.max(work, axis=0, keepdims=True), EPS) / QMAX
    k_q = jnp.clip(jnp.round(work / k_scale * jnp.sign(k)), -QMAX, QMAX)
    k_rec = jnp.where(work == 0.0, k, k_q * k_scale)

    workv = jnp.where(sink, 0.0, jnp.abs(v))
    for _ in range(K_OUT_VALS):
        rowmax = jnp.max(workv, axis=1, keepdims=True)
        workv = jnp.where(workv == rowmax, 0.0, workv)
    v_scale = jnp.maximum(jnp.max(workv, axis=1, keepdims=True), EPS) / QMAX
    v_q = jnp.clip(jnp.round(workv / v_scale * jnp.sign(v)), -QMAX, QMAX)
    v_rec = jnp.where(workv == 0.0, v, v_q * v_scale)

    k_bf = k_rec.astype(jnp.bfloat16)
    v_bf = v_rec.astype(jnp.bfloat16)

    inv_sqrt_d = 1.0 / math.sqrt(d)
    for i in range(s // BQ):
        span = (i + 1) * BQ
        qb = (q_ref[0, 0, i * BQ:span, :] * inv_sqrt_d).astype(jnp.bfloat16)
        row = i * BQ + jax.lax.broadcasted_iota(jnp.int32, (BQ, span), 0)
        col = jax.lax.broadcasted_iota(jnp.int32, (BQ, span), 1)
        scores = jax.lax.dot_general(
            qb, k_bf[:span, :], (((1,), (1,)), ((), ())),
            preferred_element_type=jnp.float32,
        )
        scores = jnp.where(col <= row, scores, NEG)
        m = jnp.max(scores, axis=1, keepdims=True)
        p = jnp.exp(scores - m)
        l = jnp.sum(p, axis=1, keepdims=True)
        acc = jax.lax.dot_general(
            p.astype(jnp.bfloat16), v_bf[:span, :], (((1,), (0,)), ((), ())),
            preferred_element_type=jnp.float32,
        )
        o_ref[0, 0, i * BQ:span, :] = acc / l


def kernel(q_tensor, k_tensor, v_tensor):
    b, h, s, d = q_tensor.shape
    spec = pl.BlockSpec((1, 1, s, d), lambda i: (0, i, 0, 0))
    out = pl.pallas_call(
        _body,
        grid=(b * h,),
        in_specs=[spec, spec, spec],
        out_specs=spec,
        out_shape=jax.ShapeDtypeStruct((b, h, s, d), jnp.float32),
        compiler_params=pltpu.CompilerParams(
            dimension_semantics=("parallel",),
        ),
    )(q_tensor, k_tensor, v_tensor)
    return out


# speedup vs baseline: 14.2014x; 1.1022x over previous
import math

import jax
import jax.numpy as jnp
from jax.experimental import pallas as pl
from jax.experimental.pallas import tpu as pltpu

SINK = 4
QMAX = 127.0
K_OUT_KEYS = 8
K_OUT_VALS = 2
EPS = 1e-8
NEG = -1e30
BQ = 512


def _body(q_ref, k_ref, v_ref, o_ref):
    k = k_ref[0, 0]
    v = v_ref[0, 0]
    s, d = k.shape
    s_iota = jax.lax.broadcasted_iota(jnp.int32, (s, d), 0)
    d_iota = jax.lax.broadcasted_iota(jnp.int32, (s, d), 1)
    sink = s_iota < SINK

    work = jnp.where(sink, 0.0, jnp.abs(k))
    for _ in range(K_OUT_KEYS):
        colmax = jnp.max(work, axis=0, keepdims=True)
        work = jnp.where(work == colmax, 0.0, work)
    k_scale = jnp.maximum(jnp.max(work, axis=0, keepdims=True), EPS) / QMAX
    k_q = jnp.clip(jnp.round(work / k_scale * jnp.sign(k)), -QMAX, QMAX)
    k_rec = jnp.where(work == 0.0, k, k_q * k_scale)

    workv = jnp.where(sink, 0.0, jnp.abs(v))
    for _ in range(K_OUT_VALS):
        rowmax = jnp.max(workv, axis=1, keepdims=True)
        workv = jnp.where(workv == rowmax, 0.0, workv)
    v_scale = jnp.maximum(jnp.max(workv, axis=1, keepdims=True), EPS) / QMAX
    v_q = jnp.clip(jnp.round(workv / v_scale * jnp.sign(v)), -QMAX, QMAX)
    v_rec = jnp.where(workv == 0.0, v, v_q * v_scale)

    k_bf = k_rec.astype(jnp.bfloat16)
    v_ext = jnp.concatenate(
        [v_rec, jnp.ones((s, 1), jnp.float32)], axis=1
    ).astype(jnp.bfloat16)

    inv_sqrt_d = 1.0 / math.sqrt(d)
    tri = (
        jax.lax.broadcasted_iota(jnp.int32, (BQ, BQ), 1)
        <= jax.lax.broadcasted_iota(jnp.int32, (BQ, BQ), 0)
    )
    for i in range(s // BQ):
        span = (i + 1) * BQ
        qb = (q_ref[0, 0, i * BQ:span, :] * inv_sqrt_d).astype(jnp.bfloat16)
        s_diag = jax.lax.dot_general(
            qb, k_bf[i * BQ:span, :], (((1,), (1,)), ((), ())),
            preferred_element_type=jnp.float32,
        )
        p_diag = jnp.exp(jnp.where(tri, s_diag, NEG)).astype(jnp.bfloat16)
        acc = jax.lax.dot_general(
            p_diag, v_ext[i * BQ:span, :], (((1,), (0,)), ((), ())),
            preferred_element_type=jnp.float32,
        )
        if i > 0:
            s_pre = jax.lax.dot_general(
                qb, k_bf[:i * BQ, :], (((1,), (1,)), ((), ())),
                preferred_element_type=jnp.float32,
            )
            p_pre = jnp.exp(s_pre).astype(jnp.bfloat16)
            acc = acc + jax.lax.dot_general(
                p_pre, v_ext[:i * BQ, :], (((1,), (0,)), ((), ())),
                preferred_element_type=jnp.float32,
            )
        o_ref[0, 0, i * BQ:span, :] = acc[:, :d] / acc[:, d:d + 1]


def kernel(q_tensor, k_tensor, v_tensor):
    b, h, s, d = q_tensor.shape
    spec = pl.BlockSpec((1, 1, s, d), lambda i: (0, i, 0, 0))
    out = pl.pallas_call(
        _body,
        grid=(b * h,),
        in_specs=[spec, spec, spec],
        out_specs=spec,
        out_shape=jax.ShapeDtypeStruct((b, h, s, d), jnp.float32),
        compiler_params=pltpu.CompilerParams(
            dimension_semantics=("parallel",),
        ),
    )(q_tensor, k_tensor, v_tensor)
    return out
